# Initial kernel scaffold; baseline (speedup 1.0000x reference)
import jax
import jax.numpy as jnp
from jax.experimental import pallas as pl


def kernel(boxes, scores, classes, window_idx):
    raise NotImplementedError("write your pallas kernel here")


# speedup vs baseline: 2.7231x; 2.7231x over previous
import jax
import jax.numpy as jnp
from jax import lax
from jax.experimental import pallas as pl

_N = 20000
_NP = 20480
_RN = 160
_PRE = 300
_CAP = 384
_DETS = 100
_BORDER = 2.0
_WIN = 1024.0
_IMG = 4096.0
_SAMP = 2.0
_CLS_OFF = 8192.0
_NMS_T = 0.5
_SCORE_T = 1e-8


def _nms_kernel(x1_ref, y1_ref, x2_ref, y2_ref, s_ref, c_ref, w_ref, out_ref):
    x1 = x1_ref[...]
    y1 = y1_ref[...]
    x2 = x2_ref[...]
    y2 = y2_ref[...]
    s = s_ref[...]
    cls = c_ref[...]
    win = w_ref[...]

    good = (x1 >= _BORDER) & (y1 >= _BORDER) \
         & (x2 < _WIN - _BORDER) & (y2 < _WIN - _BORDER)
    drone = (cls == 80.0) | (cls == 81.0)
    valid = good & drone & (s > _SCORE_T)
    s_eff = jnp.where(valid, s, 0.0)

    wx = jnp.floor(win * 0.25)
    wy = win - wx * 4.0
    offx = wx * 1024.0
    offy = wy * 1024.0
    tx1 = jnp.clip(x1 * _SAMP + offx, 0.0, _IMG)
    ty1 = jnp.clip(y1 * _SAMP + offy, 0.0, _IMG)
    tx2 = jnp.clip(x2 * _SAMP + offx, 0.0, _IMG)
    ty2 = jnp.clip(y2 * _SAMP + offy, 0.0, _IMG)

    flat = (lax.broadcasted_iota(jnp.int32, (_RN, 128), 0) * 128
            + lax.broadcasted_iota(jnp.int32, (_RN, 128), 1))
    cap_iota = lax.broadcasted_iota(jnp.int32, (1, _CAP), 1)
    zrow = jnp.zeros((1, _CAP), jnp.float32)

    def sel_body(i, carry):
        s_work, ts, ux1, uy1, ux2, uy2, uc = carry
        m = jnp.max(s_work)
        idx = jnp.min(jnp.where(s_work == m, flat, jnp.int32(_NP)))
        sel = flat == idx
        selw = sel.astype(jnp.float32)
        vx1 = jnp.sum(selw * tx1)
        vy1 = jnp.sum(selw * ty1)
        vx2 = jnp.sum(selw * tx2)
        vy2 = jnp.sum(selw * ty2)
        vc = jnp.sum(selw * cls)
        hit = (cap_iota == i).astype(jnp.float32)
        ts = ts + hit * m
        ux1 = ux1 + hit * vx1
        uy1 = uy1 + hit * vy1
        ux2 = ux2 + hit * vx2
        uy2 = uy2 + hit * vy2
        uc = uc + hit * vc
        s_work = jnp.where(sel, -1.0, s_work)
        return (s_work, ts, ux1, uy1, ux2, uy2, uc)

    carry0 = (s_eff, zrow, zrow, zrow, zrow, zrow, zrow)
    _, ts, ux1, uy1, ux2, uy2, uc = lax.fori_loop(0, _PRE, sel_body, carry0)

    co = uc * _CLS_OFF
    rx1 = ux1 + co
    ry1 = uy1 + co
    rx2 = ux2 + co
    ry2 = uy2 + co
    area = (rx2 - rx1) * (ry2 - ry1)
    keep0 = (ts > 0.0).astype(jnp.float32)

    def nms_body(i, keep):
        hm = (cap_iota == i).astype(jnp.float32)
        bx1 = jnp.sum(hm * rx1)
        by1 = jnp.sum(hm * ry1)
        bx2 = jnp.sum(hm * rx2)
        by2 = jnp.sum(hm * ry2)
        ai = jnp.sum(hm * area)
        ki = jnp.sum(hm * keep) > 0.0
        wi = jnp.clip(jnp.minimum(bx2, rx2) - jnp.maximum(bx1, rx1), 0.0, None)
        hi = jnp.clip(jnp.minimum(by2, ry2) - jnp.maximum(by1, ry1), 0.0, None)
        inter = wi * hi
        iou = inter / (ai + area - inter + 1e-9)
        sup = (iou > _NMS_T) & (cap_iota > i) & ki
        return jnp.where(sup, 0.0, keep)

    keep = lax.fori_loop(0, _PRE, nms_body, keep0)
    fs = keep * ts

    out_row = lax.broadcasted_iota(jnp.int32, (8, 128), 0)
    out_col = lax.broadcasted_iota(jnp.int32, (8, 128), 1)

    def out_body(t, carry):
        fsel, acc = carry
        m = jnp.max(fsel)
        idx = jnp.min(jnp.where(fsel == m, cap_iota, jnp.int32(_CAP)))
        hm = (cap_iota == idx).astype(jnp.float32)
        gx1 = jnp.sum(hm * ux1)
        gy1 = jnp.sum(hm * uy1)
        gx2 = jnp.sum(hm * ux2)
        gy2 = jnp.sum(hm * uy2)
        vals = jnp.where(out_row == 0, gx1,
               jnp.where(out_row == 1, gy1,
               jnp.where(out_row == 2, gx2,
               jnp.where(out_row == 3, gy2, m))))
        acc = acc + jnp.where((out_col == t) & (out_row < 5), vals, 0.0)
        fsel = jnp.where(cap_iota == idx, -1.0, fsel)
        return (fsel, acc)

    _, acc = lax.fori_loop(0, _DETS, out_body,
                           (fs, jnp.zeros((8, 128), jnp.float32)))
    out_ref[...] = acc


def kernel(boxes, scores, classes, window_idx):
    pad = _NP - _N

    def prep(v):
        return jnp.pad(v, (0, pad)).reshape(_RN, 128)

    x1 = prep(boxes[:, 0])
    y1 = prep(boxes[:, 1])
    x2 = prep(boxes[:, 2])
    y2 = prep(boxes[:, 3])
    s = prep(scores)
    c = prep(classes.astype(jnp.float32))
    w = prep(window_idx.astype(jnp.float32))

    out = pl.pallas_call(
        _nms_kernel,
        out_shape=jax.ShapeDtypeStruct((8, 128), jnp.float32),
    )(x1, y1, x2, y2, s, c, w)

    return jnp.stack([out[0, :_DETS], out[1, :_DETS], out[2, :_DETS],
                      out[3, :_DETS], out[4, :_DETS]], axis=1)


# speedup vs baseline: 2.8117x; 1.0325x over previous
import jax
import jax.numpy as jnp
from jax import lax
from jax.experimental import pallas as pl
from jax.experimental.pallas import tpu as pltpu

_N = 20000
_NP = 20480
_RN = 160
_PRE = 300
_CAP = 384
_DETS = 100
_BORDER = 2.0
_WIN = 1024.0
_IMG = 4096.0
_SAMP = 2.0
_CLS_OFF = 8192.0
_NMS_T = 0.5
_SCORE_T = 1e-8


def _nms_kernel(x1_ref, y1_ref, x2_ref, y2_ref, s_ref, c_ref, w_ref, out_ref,
                s_scr):
    x1 = x1_ref[...]
    y1 = y1_ref[...]
    x2 = x2_ref[...]
    y2 = y2_ref[...]
    s = s_ref[...]
    cls = c_ref[...]

    good = (x1 >= _BORDER) & (y1 >= _BORDER) \
         & (x2 < _WIN - _BORDER) & (y2 < _WIN - _BORDER)
    drone = (cls == 80.0) | (cls == 81.0)
    valid = good & drone & (s > _SCORE_T)
    s_scr[...] = jnp.where(valid, s, 0.0)

    flat = (lax.broadcasted_iota(jnp.int32, (_RN, 128), 0) * 128
            + lax.broadcasted_iota(jnp.int32, (_RN, 128), 1))
    lane = lax.broadcasted_iota(jnp.int32, (1, 128), 1)
    cap_iota = lax.broadcasted_iota(jnp.int32, (1, _CAP), 1)
    zrow = jnp.zeros((1, _CAP), jnp.float32)

    def sel_body(i, carry):
        ts, ux1, uy1, ux2, uy2, uc = carry
        s_work = s_scr[...]
        m = jnp.max(s_work)
        idx = jnp.min(jnp.where(s_work == m, flat, jnp.int32(_NP)))
        r = idx // 128
        c = idx - r * 128
        lm = (lane == c).astype(jnp.float32)
        vx1 = jnp.sum(x1_ref[pl.ds(r, 1), :] * lm)
        vy1 = jnp.sum(y1_ref[pl.ds(r, 1), :] * lm)
        vx2 = jnp.sum(x2_ref[pl.ds(r, 1), :] * lm)
        vy2 = jnp.sum(y2_ref[pl.ds(r, 1), :] * lm)
        vc = jnp.sum(c_ref[pl.ds(r, 1), :] * lm)
        vw = jnp.sum(w_ref[pl.ds(r, 1), :] * lm)
        wq = jnp.floor(vw * 0.25)
        offx = wq * 1024.0
        offy = (vw - wq * 4.0) * 1024.0
        hit = (cap_iota == i).astype(jnp.float32)
        ts = ts + hit * m
        ux1 = ux1 + hit * jnp.clip(vx1 * _SAMP + offx, 0.0, _IMG)
        uy1 = uy1 + hit * jnp.clip(vy1 * _SAMP + offy, 0.0, _IMG)
        ux2 = ux2 + hit * jnp.clip(vx2 * _SAMP + offx, 0.0, _IMG)
        uy2 = uy2 + hit * jnp.clip(vy2 * _SAMP + offy, 0.0, _IMG)
        uc = uc + hit * vc
        srow = s_scr[pl.ds(r, 1), :]
        s_scr[pl.ds(r, 1), :] = jnp.where(lane == c, -1.0, srow)
        return (ts, ux1, uy1, ux2, uy2, uc)

    carry0 = (zrow, zrow, zrow, zrow, zrow, zrow)
    ts, ux1, uy1, ux2, uy2, uc = lax.fori_loop(0, _PRE, sel_body, carry0)

    co = uc * _CLS_OFF
    rx1 = ux1 + co
    ry1 = uy1 + co
    rx2 = ux2 + co
    ry2 = uy2 + co
    area = (rx2 - rx1) * (ry2 - ry1)
    keep0 = (ts > 0.0).astype(jnp.float32)

    def nms_body(i, keep):
        hm = (cap_iota == i).astype(jnp.float32)
        bx1 = jnp.sum(hm * rx1)
        by1 = jnp.sum(hm * ry1)
        bx2 = jnp.sum(hm * rx2)
        by2 = jnp.sum(hm * ry2)
        ai = jnp.sum(hm * area)
        ki = jnp.sum(hm * keep) > 0.0
        wi = jnp.clip(jnp.minimum(bx2, rx2) - jnp.maximum(bx1, rx1), 0.0, None)
        hi = jnp.clip(jnp.minimum(by2, ry2) - jnp.maximum(by1, ry1), 0.0, None)
        inter = wi * hi
        iou = inter / (ai + area - inter + 1e-9)
        sup = (iou > _NMS_T) & (cap_iota > i) & ki
        return jnp.where(sup, 0.0, keep)

    keep = lax.fori_loop(0, _PRE, nms_body, keep0)
    fs = keep * ts

    out_row = lax.broadcasted_iota(jnp.int32, (8, 128), 0)
    out_col = lax.broadcasted_iota(jnp.int32, (8, 128), 1)

    def out_body(t, carry):
        fsel, acc = carry
        m = jnp.max(fsel)
        idx = jnp.min(jnp.where(fsel == m, cap_iota, jnp.int32(_CAP)))
        hm = (cap_iota == idx).astype(jnp.float32)
        gx1 = jnp.sum(hm * ux1)
        gy1 = jnp.sum(hm * uy1)
        gx2 = jnp.sum(hm * ux2)
        gy2 = jnp.sum(hm * uy2)
        vals = jnp.where(out_row == 0, gx1,
               jnp.where(out_row == 1, gy1,
               jnp.where(out_row == 2, gx2,
               jnp.where(out_row == 3, gy2, m))))
        acc = acc + jnp.where((out_col == t) & (out_row < 5), vals, 0.0)
        fsel = jnp.where(cap_iota == idx, -1.0, fsel)
        return (fsel, acc)

    _, acc = lax.fori_loop(0, _DETS, out_body,
                           (fs, jnp.zeros((8, 128), jnp.float32)))
    out_ref[...] = acc


def kernel(boxes, scores, classes, window_idx):
    pad = _NP - _N

    def prep(v):
        return jnp.pad(v, (0, pad)).reshape(_RN, 128)

    x1 = prep(boxes[:, 0])
    y1 = prep(boxes[:, 1])
    x2 = prep(boxes[:, 2])
    y2 = prep(boxes[:, 3])
    s = prep(scores)
    c = prep(classes.astype(jnp.float32))
    w = prep(window_idx.astype(jnp.float32))

    out = pl.pallas_call(
        _nms_kernel,
        out_shape=jax.ShapeDtypeStruct((8, 128), jnp.float32),
        scratch_shapes=[pltpu.VMEM((_RN, 128), jnp.float32)],
    )(x1, y1, x2, y2, s, c, w)

    return jnp.stack([out[0, :_DETS], out[1, :_DETS], out[2, :_DETS],
                      out[3, :_DETS], out[4, :_DETS]], axis=1)
